# baseline (speedup 1.0000x reference)
import functools

import jax
import jax.numpy as jnp
from jax import lax
from jax.experimental import pallas as pl
from jax.experimental.pallas import tpu as pltpu
from jax.experimental.pallas import tpu_sc as plsc


@functools.lru_cache(maxsize=None)
def _build(M, K, V):
    scale = 1.0 / M
    mesh = plsc.ScalarSubcoreMesh(axis_name="c", num_cores=1)

    @functools.partial(
        pl.kernel,
        mesh=mesh,
        compiler_params=pltpu.CompilerParams(
            needs_layout_passes=False,
            disable_bounds_checks=True,
            disable_semaphore_checks=True,
            skip_device_barrier=True,
        ),
        out_type=jax.ShapeDtypeStruct((M, V), jnp.float32),
        scratch_types=[
            pltpu.SMEM((M, K, V), jnp.float32),
            pltpu.SMEM((2, V), jnp.int32),
            pltpu.SMEM((M, V), jnp.float32),
            pltpu.SMEM((M, V), jnp.float32),
            pltpu.SemaphoreType.DMA,
            pltpu.SemaphoreType.DMA,
        ],
    )
    def scs_kernel(
        ivf_hbm, idx_hbm, wout_hbm, out_hbm, ivf_s, idx_s, wout_s, out_s, sem, sem_w
    ):
        c1 = pltpu.make_async_copy(ivf_hbm, ivf_s, sem)
        c2 = pltpu.make_async_copy(idx_hbm, idx_s, sem)
        c3 = pltpu.make_async_copy(wout_hbm, wout_s, sem_w)
        c1.start()
        c2.start()
        c3.start()
        c1.wait()
        c2.wait()
        prods = []
        for v in range(V):
            i0 = idx_s[0, v]
            i1 = idx_s[1, v]
            for m in range(M):
                prods.append((m, v, scale * ivf_s[m, i0, v] * ivf_s[m, i1, v]))
        c3.wait()
        for m, v, p in prods:
            out_s[m, v] = p * wout_s[m, v]
        pltpu.sync_copy(out_s, out_hbm)

    return scs_kernel


def kernel(num_M, num_VN, IVF, VN_index, wout):
    M, K, V = IVF.shape
    return _build(M, K, V)(
        IVF.astype(jnp.float32),
        VN_index.astype(jnp.int32),
        wout.astype(jnp.float32),
    )
